# Initial kernel scaffold; baseline (speedup 1.0000x reference)
import jax
import jax.numpy as jnp
from jax.experimental import pallas as pl


def kernel(x, edge_index, edge_weight, W1, b1, W2, b2, Wl, bl):
    raise NotImplementedError("write your pallas kernel here")


# speedup vs baseline: 16.6866x; 16.6866x over previous
import functools

import jax
import jax.numpy as jnp
from jax import lax
from jax.experimental import pallas as pl
from jax.experimental.pallas import tpu as pltpu
from jax.experimental.pallas import tpu_sc as plsc

N = 10000
NP = 10240
E = 320000
NC = 2
NS = 16
NW = NC * NS
CH = 128
NCHK = 79
ET = NCHK * CH
EP = NW * ET
F_IN = 128
F_H = 64
RPS = NP // NS
R = 1280

_f32 = jnp.float32
_mesh = plsc.VectorSubcoreMesh(core_axis_name="c", subcore_axis_name="s")


def _deg_body(dst_hbm, w_hbm, out_hbm, dstb, wb, zb, dacc):
    c = lax.axis_index("c")
    s = lax.axis_index("s")
    wid = c * NS + s
    pltpu.sync_copy(dst_hbm.at[wid], dstb)
    pltpu.sync_copy(w_hbm.at[wid], wb)

    zero16 = jnp.zeros((16,), _f32)

    def zloop(r, carry):
        zb[pl.ds(r * 16, 16)] = zero16
        return carry

    lax.fori_loop(0, RPS // 16, zloop, 0)
    base = s * RPS
    pltpu.sync_copy(zb, dacc.at[pl.ds(base, RPS)])
    plsc.subcore_barrier()

    def chunk(j, carry):
        pltpu.sync_copy(wb.at[j], dacc.at[dstb.at[j]], add=True)
        return carry

    lax.fori_loop(0, NCHK, chunk, 0)
    plsc.subcore_barrier()
    pltpu.sync_copy(dacc.at[pl.ds(base, RPS)], out_hbm.at[c, pl.ds(base, RPS)])


_deg_call = pl.kernel(
    _deg_body,
    out_type=jax.ShapeDtypeStruct((NC, NP), _f32),
    mesh=_mesh,
    scratch_types=[
        pltpu.VMEM((NCHK, CH), jnp.int32),
        pltpu.VMEM((NCHK, CH), _f32),
        pltpu.VMEM((RPS,), _f32),
        pltpu.VMEM_SHARED((NP,), _f32),
    ],
)


def _agg_body(h_hbm, src_hbm, dst_hbm, w_hbm, dinv_hbm, out_hbm,
              srcb, dstb, wb, dinvb, rows, acc, sem):
    c = lax.axis_index("c")
    s = lax.axis_index("s")
    wid = c * NS + s
    pltpu.sync_copy(src_hbm.at[wid], srcb)
    pltpu.sync_copy(dst_hbm.at[wid], dstb)
    pltpu.sync_copy(w_hbm.at[wid], wb)
    pltpu.sync_copy(dinv_hbm, dinvb)

    zero16 = jnp.zeros((16,), _f32)

    def zloop(r, carry):
        for k in range(F_H // 16):
            rows[r, pl.ds(k * 16, 16)] = zero16
        return carry

    lax.fori_loop(0, CH, zloop, 0)
    base = s * RPS
    for t in range(RPS // CH):
        pltpu.sync_copy(rows, acc.at[pl.ds(base + t * CH, CH)])
    plsc.subcore_barrier()

    def chunk(j, carry):
        pltpu.async_copy(h_hbm.at[srcb.at[j]], rows, sem).wait()
        for g in range(CH // 16):
            sl = pl.ds(g * 16, 16)
            srcv = srcb[j, sl]
            dstv = dstb[j, sl]
            wv = wb[j, sl]
            norm = (plsc.load_gather(dinvb, [srcv]) * wv
                    * plsc.load_gather(dinvb, [dstv]))
            for l in range(16):
                scale = jnp.full((16,), norm[l], _f32)
                r = g * 16 + l
                for k in range(F_H // 16):
                    fsl = pl.ds(k * 16, 16)
                    rows[r, fsl] = rows[r, fsl] * scale
        pltpu.sync_copy(rows, acc.at[dstb.at[j]], add=True)
        return carry

    lax.fori_loop(0, NCHK, chunk, 0)
    plsc.subcore_barrier()
    pltpu.sync_copy(acc.at[pl.ds(base, RPS)],
                    out_hbm.at[c, pl.ds(base, RPS)])


_agg_call = pl.kernel(
    _agg_body,
    out_type=jax.ShapeDtypeStruct((NC, NP, F_H), _f32),
    mesh=_mesh,
    compiler_params=pltpu.CompilerParams(needs_layout_passes=False,
                                         use_tc_tiling_on_sc=False),
    scratch_types=[
        pltpu.VMEM((NCHK, CH), jnp.int32),
        pltpu.VMEM((NCHK, CH), jnp.int32),
        pltpu.VMEM((NCHK, CH), _f32),
        pltpu.VMEM((NP,), _f32),
        pltpu.VMEM((CH, F_H), _f32),
        pltpu.VMEM_SHARED((NP, F_H), _f32),
        pltpu.SemaphoreType.DMA,
    ],
)


def _tc1_body(pt_ref, x_ref, w1_ref, h1_ref, dinv_ref, invdeg_ref):
    p = pt_ref[...]
    deg = p[:, 0:1] + p[:, 1:2] + 1.0
    invdeg_ref[...] = 1.0 / deg
    dinv_ref[...] = lax.rsqrt(deg)
    h1_ref[...] = jnp.dot(x_ref[...], w1_ref[...],
                          preferred_element_type=_f32)


_tc1 = pl.pallas_call(
    _tc1_body,
    grid=(NP // R,),
    in_specs=[
        pl.BlockSpec((R, 2), lambda i: (i, 0)),
        pl.BlockSpec((R, F_IN), lambda i: (i, 0)),
        pl.BlockSpec((F_IN, F_H), lambda i: (0, 0)),
    ],
    out_specs=[
        pl.BlockSpec((R, F_H), lambda i: (i, 0)),
        pl.BlockSpec((R, 1), lambda i: (i, 0)),
        pl.BlockSpec((R, 1), lambda i: (i, 0)),
    ],
    out_shape=[
        jax.ShapeDtypeStruct((NP, F_H), _f32),
        jax.ShapeDtypeStruct((NP, 1), _f32),
        jax.ShapeDtypeStruct((NP, 1), _f32),
    ],
)


def _combine_mm_body(s_ref, h_ref, invdeg_ref, b_ref, w_ref, out_ref):
    sarr = s_ref[...]
    z = sarr[0] + sarr[1] + h_ref[...] * invdeg_ref[...] + b_ref[...]
    z = jnp.maximum(z, 0.0)
    out_ref[...] = jnp.dot(z, w_ref[...], preferred_element_type=_f32)


def _make_combine_mm(n_out):
    return pl.pallas_call(
        _combine_mm_body,
        grid=(NP // R,),
        in_specs=[
            pl.BlockSpec((NC, R, F_H), lambda i: (0, i, 0)),
            pl.BlockSpec((R, F_H), lambda i: (i, 0)),
            pl.BlockSpec((R, 1), lambda i: (i, 0)),
            pl.BlockSpec((1, F_H), lambda i: (0, 0)),
            pl.BlockSpec((F_H, n_out), lambda i: (0, 0)),
        ],
        out_specs=pl.BlockSpec((R, n_out), lambda i: (i, 0)),
        out_shape=jax.ShapeDtypeStruct((NP, n_out), _f32),
    )


_tc2 = _make_combine_mm(F_H)


def _head_body(s_ref, h_ref, invdeg_ref, b_ref, wl_ref, bl_ref, out_ref):
    sarr = s_ref[...]
    z = sarr[0] + sarr[1] + h_ref[...] * invdeg_ref[...] + b_ref[...]
    z = jnp.maximum(z, 0.0)
    out_ref[...] = jnp.dot(z, wl_ref[...],
                           preferred_element_type=_f32) + bl_ref[...]


_tc3 = pl.pallas_call(
    _head_body,
    grid=(NP // R,),
    in_specs=[
        pl.BlockSpec((NC, R, F_H), lambda i: (0, i, 0)),
        pl.BlockSpec((R, F_H), lambda i: (i, 0)),
        pl.BlockSpec((R, 1), lambda i: (i, 0)),
        pl.BlockSpec((1, F_H), lambda i: (0, 0)),
        pl.BlockSpec((F_H, 1), lambda i: (0, 0)),
        pl.BlockSpec((1, 1), lambda i: (0, 0)),
    ],
    out_specs=pl.BlockSpec((R, 1), lambda i: (i, 0)),
    out_shape=jax.ShapeDtypeStruct((NP, 1), _f32),
)


def kernel(x, edge_index, edge_weight, W1, b1, W2, b2, Wl, bl):
    src = edge_index[0]
    dst = edge_index[1]
    pad = EP - E
    src_p = jnp.concatenate(
        [src, jnp.zeros((pad,), src.dtype)]).reshape(NW, NCHK, CH)
    dst_p = jnp.concatenate(
        [dst, jnp.zeros((pad,), dst.dtype)]).reshape(NW, NCHK, CH)
    w_p = jnp.concatenate(
        [edge_weight, jnp.zeros((pad,), edge_weight.dtype)]
    ).reshape(NW, NCHK, CH)
    x_p = jnp.pad(x, ((0, NP - N), (0, 0)))

    deg_parts = _deg_call(dst_p, w_p)
    h1, dinv_col, invdeg_col = _tc1(deg_parts.T, x_p, W1)
    dinv = dinv_col.reshape(NP)
    s1 = _agg_call(h1, src_p, dst_p, w_p, dinv)
    h2 = _tc2(s1, h1, invdeg_col, b1.reshape(1, F_H), W2)
    s2 = _agg_call(h2, src_p, dst_p, w_p, dinv)
    out_col = _tc3(s2, h2, invdeg_col, b2.reshape(1, F_H),
                   Wl, bl.reshape(1, 1))
    return out_col[:N, 0]
